# Initial kernel scaffold; baseline (speedup 1.0000x reference)
import jax
import jax.numpy as jnp
from jax.experimental import pallas as pl


def kernel(lt, inputs):
    raise NotImplementedError("write your pallas kernel here")


# speedup vs baseline: 2.0728x; 2.0728x over previous
import functools

import jax
import jax.numpy as jnp
from jax import lax
from jax.experimental import pallas as pl
from jax.experimental.pallas import tpu as pltpu
from jax.experimental.pallas import tpu_sc as plsc

SIZE = 1000000
DIM = 32
B = 16384
N = 50
MAXNORM = 1.0 - 1e-5
EPS = 1e-7

NC = 2
NS = 16
NW = NC * NS
BPW = B // NW
CB = 16
CHUNKS = BPW // CB
ROWS = CB * N
GS = 80
NG = ROWS // GS


def _sc_gather_reduce(lt, idx_flat):
  mesh = plsc.VectorSubcoreMesh(core_axis_name="c", subcore_axis_name="s")

  @functools.partial(
      pl.kernel,
      out_type=[
          jax.ShapeDtypeStruct((B * N,), jnp.float32),
          jax.ShapeDtypeStruct((B * (N - 1),), jnp.float32),
      ],
      mesh=mesh,
      compiler_params=pltpu.CompilerParams(needs_layout_passes=False,
                                           use_tc_tiling_on_sc=False),
      scratch_types=[
          pltpu.VMEM((ROWS,), jnp.int32),
          pltpu.VMEM((ROWS, DIM), jnp.float32),
          pltpu.VMEM((ROWS,), jnp.float32),
          pltpu.VMEM((CB * (N - 1),), jnp.float32),
          pltpu.SemaphoreType.DMA,
      ],
  )
  def k(lt_hbm, idx_hbm, nsq_hbm, dot_hbm, idx_v, rows_v, nsq_v, dot_v, sem):
    wid = lax.axis_index("s") * NC + lax.axis_index("c")
    lane = lax.iota(jnp.int32, 16)
    mask15 = lane == 15

    def _put(ref, pos, vec):
      plsc.store_scatter(ref, [jnp.full((16,), pos, jnp.int32)], vec,
                         mask=mask15)

    def chunk_body(c, carry):
      b0 = wid * BPW + c * CB
      pltpu.sync_copy(idx_hbm.at[pl.ds(b0 * N, ROWS)], idx_v)
      handles = []
      for g in range(NG):
        handles.append(pltpu.async_copy(
            lt_hbm.at[idx_v.at[pl.ds(g * GS, GS)]],
            rows_v.at[pl.ds(g * GS, GS)], sem))
      for h in handles:
        h.wait()

      def batch_body(i, carry2):
        r0 = i * N
        u0 = rows_v[r0, pl.ds(0, 16)]
        u1 = rows_v[r0, pl.ds(16, 16)]
        _put(nsq_v, r0, plsc.cumsum(u0 * u0 + u1 * u1))

        def j_body(j, carry3):
          v0 = rows_v[r0 + j, pl.ds(0, 16)]
          v1 = rows_v[r0 + j, pl.ds(16, 16)]
          _put(nsq_v, r0 + j, plsc.cumsum(v0 * v0 + v1 * v1))
          _put(dot_v, i * (N - 1) + j - 1, plsc.cumsum(u0 * v0 + u1 * v1))
          return carry3

        lax.fori_loop(1, N, j_body, 0)
        return carry2

      lax.fori_loop(0, CB, batch_body, 0)
      pltpu.sync_copy(nsq_v, nsq_hbm.at[pl.ds(b0 * N, ROWS)])
      pltpu.sync_copy(dot_v, dot_hbm.at[pl.ds(b0 * (N - 1), CB * (N - 1))])
      return carry

    lax.fori_loop(0, CHUNKS, chunk_body, 0)

  return k(lt, idx_flat)


def _finish(nsq, dots):
  BB = 1024

  def body(nsq_ref, dot_ref, o_ref):
    n = nsq_ref[...]
    norm = jnp.sqrt(n)
    scale = jnp.where(norm > MAXNORM, MAXNORM / jnp.maximum(norm, 1e-12), 1.0)
    n0 = n[:, 0:1]
    s0 = scale[:, 0:1]
    nj = n[:, 1:]
    sj = scale[:, 1:]
    d = dot_ref[...]
    sq_u = (s0 * s0) * n0
    sq_v = (sj * sj) * nj
    cross = ((2.0 * s0) * sj) * d
    sqdist = sq_u + sq_v - cross
    denom = (1.0 - sq_u) * (1.0 - sq_v) + EPS
    x = 1.0 + 2.0 * sqdist / denom
    x = jnp.maximum(x, 1.0 + EPS)
    o_ref[...] = jnp.log(x + jnp.sqrt((x - 1.0) * (x + 1.0)))

  return pl.pallas_call(
      body,
      grid=(B // BB,),
      in_specs=[
          pl.BlockSpec((BB, N), lambda i: (i, 0)),
          pl.BlockSpec((BB, N - 1), lambda i: (i, 0)),
      ],
      out_specs=pl.BlockSpec((BB, N - 1), lambda i: (i, 0)),
      out_shape=jax.ShapeDtypeStruct((B, N - 1), jnp.float32),
  )(nsq, dots)


def kernel(lt, inputs):
  idx_flat = inputs.reshape(-1).astype(jnp.int32)
  nsq, dots = _sc_gather_reduce(lt, idx_flat)
  return _finish(nsq.reshape(B, N), dots.reshape(B, N - 1))
